# baseline (speedup 1.0000x reference)
import functools

import jax
import jax.numpy as jnp
from jax import lax
from jax.experimental import pallas as pl
from jax.experimental.pallas import tpu as pltpu
from jax.experimental.pallas import tpu_sc as plsc


def _sc_gather(table, idx):
    V, D = table.shape
    B = idx.shape[0]
    info = plsc.get_sparse_core_info()
    NC, NS = info.num_cores, info.num_subcores
    NW = NC * NS
    assert B % NW == 0
    b_per_w = B // NW
    mesh = plsc.VectorSubcoreMesh(core_axis_name="c", subcore_axis_name="s")

    @functools.partial(
        pl.kernel,
        mesh=mesh,
        out_type=jax.ShapeDtypeStruct((B, D), jnp.float32),
        scratch_types=[
            pltpu.VMEM((b_per_w,), jnp.int32),
            pltpu.VMEM((b_per_w, D), jnp.float32),
            pltpu.SemaphoreType.DMA,
        ],
        compiler_params=pltpu.CompilerParams(use_tc_tiling_on_sc=False),
    )
    def k(table_hbm, idx_hbm, out_hbm, idx_v, rows_v, sem):
        wid = lax.axis_index("s") * NC + lax.axis_index("c")
        base = wid * b_per_w
        pltpu.sync_copy(idx_hbm.at[pl.ds(base, b_per_w)], idx_v)
        pltpu.async_copy(table_hbm.at[idx_v], rows_v, sem).wait()
        pltpu.sync_copy(rows_v, out_hbm.at[pl.ds(base, b_per_w)])

    return k(table, idx)


def _proj_body(flat_ref, w_ref, b_ref, out_ref):
    out_ref[...] = (
        lax.dot_general(
            flat_ref[...],
            w_ref[...],
            dimension_numbers=(((1,), (1,)), ((), ())),
            preferred_element_type=jnp.float32,
        )
        + b_ref[...]
    )


def _projection(flat, W, b2d, vblk):
    B, K = flat.shape
    V = W.shape[0]
    nblk = (V + vblk - 1) // vblk
    return pl.pallas_call(
        _proj_body,
        grid=(nblk,),
        in_specs=[
            pl.BlockSpec((B, K), lambda j: (0, 0)),
            pl.BlockSpec((vblk, K), lambda j: (j, 0)),
            pl.BlockSpec((1, vblk), lambda j: (0, j)),
        ],
        out_specs=pl.BlockSpec((B, vblk), lambda j: (0, j)),
        out_shape=jax.ShapeDtypeStruct((B, V), jnp.float32),
    )(flat, W, b2d)


def kernel(inputs, emb_table, W, b):
    api_seq = inputs[0]
    B, N = api_seq.shape
    D = emb_table.shape[1]
    idx = api_seq.reshape(B * N)
    rows = _sc_gather(emb_table, idx)
    flat = rows.reshape(B, N * D)
    out = _projection(flat, W, b.reshape(1, -1), vblk=2048)
    return out


# speedup vs baseline: 1.0011x; 1.0011x over previous
import functools

import jax
import jax.numpy as jnp
from jax import lax
from jax.experimental import pallas as pl
from jax.experimental.pallas import tpu as pltpu
from jax.experimental.pallas import tpu_sc as plsc


def _sc_gather(table, idx):
    V, D = table.shape
    B = idx.shape[0]
    info = plsc.get_sparse_core_info()
    NC, NS = info.num_cores, info.num_subcores
    NW = NC * NS
    assert B % NW == 0
    b_per_w = B // NW
    mesh = plsc.VectorSubcoreMesh(core_axis_name="c", subcore_axis_name="s")

    @functools.partial(
        pl.kernel,
        mesh=mesh,
        out_type=jax.ShapeDtypeStruct((B, D), jnp.float32),
        scratch_types=[
            pltpu.VMEM((b_per_w,), jnp.int32),
            pltpu.VMEM((b_per_w, D), jnp.float32),
            pltpu.SemaphoreType.DMA,
        ],
        compiler_params=pltpu.CompilerParams(use_tc_tiling_on_sc=False),
    )
    def k(table_hbm, idx_hbm, out_hbm, idx_v, rows_v, sem):
        wid = lax.axis_index("s") * NC + lax.axis_index("c")
        base = wid * b_per_w
        pltpu.sync_copy(idx_hbm.at[pl.ds(base, b_per_w)], idx_v)
        pltpu.async_copy(table_hbm.at[idx_v], rows_v, sem).wait()
        pltpu.sync_copy(rows_v, out_hbm.at[pl.ds(base, b_per_w)])

    return k(table, idx)


def _proj_body(flat_ref, w_ref, b_ref, out_ref):
    out_ref[...] = (
        lax.dot_general(
            flat_ref[...].astype(jnp.bfloat16),
            w_ref[...].astype(jnp.bfloat16),
            dimension_numbers=(((1,), (1,)), ((), ())),
            preferred_element_type=jnp.float32,
        )
        + b_ref[...]
    )


def _projection(flat, W, b2d, vblk):
    B, K = flat.shape
    V = W.shape[0]
    nblk = (V + vblk - 1) // vblk
    return pl.pallas_call(
        _proj_body,
        grid=(nblk,),
        in_specs=[
            pl.BlockSpec((B, K), lambda j: (0, 0)),
            pl.BlockSpec((vblk, K), lambda j: (j, 0)),
            pl.BlockSpec((1, vblk), lambda j: (0, j)),
        ],
        out_specs=pl.BlockSpec((B, vblk), lambda j: (0, j)),
        out_shape=jax.ShapeDtypeStruct((B, V), jnp.float32),
    )(flat, W, b2d)


def kernel(inputs, emb_table, W, b):
    api_seq = inputs[0]
    B, N = api_seq.shape
    D = emb_table.shape[1]
    idx = api_seq.reshape(B * N)
    rows = _sc_gather(emb_table, idx)
    flat = rows.reshape(B, N * D)
    out = _projection(flat, W, b.reshape(1, -1), vblk=2048)
    return out
